# baseline (speedup 1.0000x reference)
import jax
import jax.numpy as jnp
from jax import lax
from jax.experimental import pallas as pl
from jax.experimental.pallas import tpu as pltpu
from jax.experimental.pallas import tpu_sc as plsc

NC = 2
NS = 16
NW = NC * NS
L = 16
C = 80
D = 128
DW = D // 2


def _sc_body(z_hbm, src_hbm, dst_hbm, out_hbm,
             z_sh, idx_s, idx_d, rows_sa, rows_da, rows_sb, rows_db,
             rows_sc, rows_dc, out_v, tr_v, sem_a, sem_b, sem_c):
    wid = lax.axis_index("s") * NC + lax.axis_index("c")
    sid = lax.axis_index("s")
    n_rows = z_hbm.shape[0]
    per_w = src_hbm.shape[0] // NW
    n_chunks = per_w // C
    base_w = wid * per_w
    lane = lax.iota(jnp.int32, L)

    r_per_s = n_rows // NS
    soff = sid * r_per_s
    pltpu.sync_copy(z_hbm.at[pl.ds(soff, r_per_s)],
                    z_sh.at[pl.ds(soff, r_per_s)])

    pltpu.sync_copy(src_hbm.at[pl.ds(base_w, per_w)], idx_s)
    pltpu.sync_copy(dst_hbm.at[pl.ds(base_w, per_w)], idx_d)
    plsc.subcore_barrier()

    def issue(c, rows_s, rows_d, sem):
        off = pl.multiple_of(c * C, C)
        pltpu.async_copy(z_sh.at[idx_s.at[pl.ds(off, C)]], rows_s, sem)
        pltpu.async_copy(z_sh.at[idx_d.at[pl.ds(off, C)]], rows_d, sem)

    def wait(c, rows_s, rows_d, sem):
        off = pl.multiple_of(c * C, C)
        pltpu.make_async_copy(z_sh.at[idx_s.at[pl.ds(off, C)]], rows_s, sem).wait()
        pltpu.make_async_copy(z_sh.at[idx_d.at[pl.ds(off, C)]], rows_d, sem).wait()

    def compute(c, rows_s, rows_d):
        def group_body(g, carry):
            sums = []
            for e_loc in range(L):
                e = g * L + e_loc
                parts = []
                for k in range(DW // L):
                    svec = plsc.bitcast(rows_s[e, pl.ds(k * L, L)], jnp.bfloat16)
                    dvec = plsc.bitcast(rows_d[e, pl.ds(k * L, L)], jnp.bfloat16)
                    pe, po = plsc.unpack(svec * dvec,
                                         format=plsc.PackFormat.INTERLEAVED)
                    parts.append(pe + po)
                while len(parts) > 1:
                    parts = [a + b for a, b in zip(parts[::2], parts[1::2])]
                sums.append(parts[0])
            for e_loc in range(L):
                plsc.store_scatter(tr_v, [lane * 17 + e_loc], sums[e_loc])
            cols = [plsc.load_gather(tr_v, [lane + l * 17]) for l in range(L)]
            while len(cols) > 1:
                cols = [a + b for a, b in zip(cols[::2], cols[1::2])]
            out_v[pl.ds(c * C + g * L, L)] = cols[0]
            return carry

        lax.fori_loop(0, C // L, group_body, 0)

    slots = ((rows_sa, rows_da, sem_a),
             (rows_sb, rows_db, sem_b),
             (rows_sc, rows_dc, sem_c))
    issue(0, *slots[0])
    issue(1, *slots[1])

    def tri_body(j, carry):
        c = 3 * j
        for t in range(3):
            wait(c + t, *slots[t])
            issue(c + t + 2, *slots[(t + 2) % 3])
            compute(c + t, slots[t][0], slots[t][1])
        return carry

    lax.fori_loop(0, (n_chunks - 2) // 3, tri_body, 0)
    wait(n_chunks - 2, *slots[0])
    compute(n_chunks - 2, slots[0][0], slots[0][1])
    wait(n_chunks - 1, *slots[1])
    compute(n_chunks - 1, slots[1][0], slots[1][1])

    pltpu.sync_copy(out_v, out_hbm.at[pl.ds(base_w, per_w)])


def kernel(z, edge_index):
    n_edges = edge_index.shape[1]
    per_w = n_edges // NW
    assert n_edges % (NW * C) == 0 and z.shape[1] == D
    assert (per_w // C) % 3 == 2
    assert z.shape[0] % NS == 0
    ei = edge_index.astype(jnp.int32)
    src = ei[0]
    dst = ei[1]
    zb = z.astype(jnp.bfloat16)
    zp = jax.lax.bitcast_convert_type(
        zb.reshape(z.shape[0], DW, 2), jnp.int32)

    mesh = plsc.VectorSubcoreMesh(core_axis_name="c", subcore_axis_name="s")
    f = pl.kernel(
        _sc_body,
        out_type=jax.ShapeDtypeStruct((n_edges,), jnp.float32),
        mesh=mesh,
        scratch_types=[
            pltpu.VMEM_SHARED((z.shape[0], DW), jnp.int32),
            pltpu.VMEM((per_w,), jnp.int32),
            pltpu.VMEM((per_w,), jnp.int32),
            pltpu.VMEM((C, DW), jnp.int32),
            pltpu.VMEM((C, DW), jnp.int32),
            pltpu.VMEM((C, DW), jnp.int32),
            pltpu.VMEM((C, DW), jnp.int32),
            pltpu.VMEM((C, DW), jnp.int32),
            pltpu.VMEM((C, DW), jnp.int32),
            pltpu.VMEM((per_w,), jnp.float32),
            pltpu.VMEM((L * 17,), jnp.float32),
            pltpu.SemaphoreType.DMA,
            pltpu.SemaphoreType.DMA,
            pltpu.SemaphoreType.DMA,
        ],
        compiler_params=pltpu.CompilerParams(needs_layout_passes=False,
                                             use_tc_tiling_on_sc=False),
    )
    return f(zp, src, dst)
